# Initial kernel scaffold; baseline (speedup 1.0000x reference)
import jax
import jax.numpy as jnp
from jax.experimental import pallas as pl


def kernel(x, edge_index, W1, b1, t1, W2, b2, t2):
    raise NotImplementedError("write your pallas kernel here")


# speedup vs baseline: 2.7332x; 2.7332x over previous
import functools

import jax
import jax.numpy as jnp
from jax import lax
from jax.experimental import pallas as pl
from jax.experimental.pallas import tpu as pltpu
from jax.experimental.pallas import tpu_sc as plsc

NC = 2
NS = 16
NW = NC * NS
LANES = 16


def _edge_kernels(n_nodes, e_pad_rows, full_rows, rem_groups, epw):
    R = e_pad_rows
    N = n_nodes
    TBL = N * N
    PADS = R * 128 - epw

    mesh = plsc.VectorSubcoreMesh(core_axis_name="c", subcore_axis_name="s")

    def compute_keys(wid, es_v, ed_v, key_v):
        iota = lax.iota(jnp.int32, LANES)

        def row_body(r, carry):
            for j in range(8):
                col = j * LANES
                a = es_v[r, pl.ds(col, LANES)]
                b = ed_v[r, pl.ds(col, LANES)]
                lo = jnp.minimum(a, b)
                hi = jnp.maximum(a, b)
                key_v[r, pl.ds(col, LANES)] = lo * N + hi
            return carry

        lax.fori_loop(0, full_rows, row_body, None)
        r = full_rows
        for j in range(8):
            col = j * LANES
            if j < rem_groups:
                a = es_v[r, pl.ds(col, LANES)]
                b = ed_v[r, pl.ds(col, LANES)]
                lo = jnp.minimum(a, b)
                hi = jnp.maximum(a, b)
                key_v[r, pl.ds(col, LANES)] = lo * N + hi
            else:
                p0 = wid * PADS + (j - rem_groups) * LANES + 1
                key_v[r, pl.ds(col, LANES)] = (iota + p0) * N

    @functools.partial(
        pl.kernel,
        out_type=jax.ShapeDtypeStruct((TBL,), jnp.int32),
        mesh=mesh,
        scratch_types=[
            pltpu.VMEM((R, 128), jnp.int32),
            pltpu.VMEM((R, 128), jnp.int32),
            pltpu.VMEM((R, 128), jnp.int32),
            pltpu.VMEM((R, 128), jnp.int32),
            pltpu.SemaphoreType.DMA,
        ],
    )
    def scatter_ids(src_hbm, dst_hbm, table_out, es_v, ed_v, key_v, id_v, sem):
        c = lax.axis_index("c")
        s = lax.axis_index("s")
        wid = s * NC + c
        pltpu.sync_copy(src_hbm.at[wid], es_v)
        pltpu.sync_copy(dst_hbm.at[wid], ed_v)
        compute_keys(wid, es_v, ed_v, key_v)
        iota = lax.iota(jnp.int32, LANES)
        base_id = wid * (R * 128)

        def id_body(r, carry):
            for j in range(8):
                col = j * LANES
                id_v[r, pl.ds(col, LANES)] = base_id + r * 128 + col + iota
            return carry

        lax.fori_loop(0, R, id_body, None)

        def sc_body(r, carry):
            pltpu.async_copy(id_v.at[r], table_out.at[key_v.at[r]], sem).wait()
            return carry

        lax.fori_loop(0, R, sc_body, None)

    NPAD = ((N + 10 * LANES * NS - 1) // (10 * LANES * NS)) * (10 * LANES * NS)
    ZCH = NPAD // NS

    @functools.partial(
        pl.kernel,
        out_type=jax.ShapeDtypeStruct((NC, NPAD), jnp.float32),
        mesh=mesh,
        scratch_types=[
            pltpu.VMEM((R, 128), jnp.int32),
            pltpu.VMEM((R, 128), jnp.int32),
            pltpu.VMEM((R, 128), jnp.int32),
            pltpu.VMEM((R, 128), jnp.int32),
            pltpu.VMEM((R, 128), jnp.float32),
            pltpu.VMEM((ZCH,), jnp.float32),
            pltpu.VMEM_SHARED((NPAD,), jnp.float32),
            pltpu.SemaphoreType.DMA,
        ],
    )
    def count_winners(table_hbm, src_hbm, dst_hbm, deg_out,
                      es_v, ed_v, key_v, got_v, w_v, z_v, deg_sh, sem):
        c = lax.axis_index("c")
        s = lax.axis_index("s")
        wid = s * NC + c
        pltpu.sync_copy(src_hbm.at[wid], es_v)
        pltpu.sync_copy(dst_hbm.at[wid], ed_v)
        compute_keys(wid, es_v, ed_v, key_v)

        def z_body(i, carry):
            z_v[pl.ds(i * LANES, LANES)] = jnp.zeros((LANES,), jnp.float32)
            return carry

        lax.fori_loop(0, ZCH // LANES, z_body, None)
        pltpu.sync_copy(z_v, deg_sh.at[pl.ds(s * ZCH, ZCH)])

        def g_body(r, carry):
            pltpu.async_copy(table_hbm.at[key_v.at[r]], got_v.at[r], sem).wait()
            return carry

        lax.fori_loop(0, R, g_body, None)

        iota = lax.iota(jnp.int32, LANES)
        base_id = wid * (R * 128)
        one = jnp.full((LANES,), 1.0, jnp.float32)
        zero = jnp.zeros((LANES,), jnp.float32)

        def c_body(r, carry):
            for j in range(8):
                col = j * LANES
                myid = base_id + r * 128 + col + iota
                w_v[r, pl.ds(col, LANES)] = jnp.where(
                    got_v[r, pl.ds(col, LANES)] == myid, one, zero)
            return carry

        lax.fori_loop(0, full_rows, c_body, None)
        r = full_rows
        for j in range(8):
            col = j * LANES
            if j < rem_groups:
                myid = base_id + r * 128 + col + iota
                w_v[r, pl.ds(col, LANES)] = jnp.where(
                    got_v[r, pl.ds(col, LANES)] == myid, one, zero)
            else:
                w_v[r, pl.ds(col, LANES)] = zero

        plsc.subcore_barrier()

        def a_body(r, carry):
            pltpu.sync_copy(w_v.at[r], deg_sh.at[es_v.at[r]], add=True)
            pltpu.sync_copy(w_v.at[r], deg_sh.at[ed_v.at[r]], add=True)
            return carry

        lax.fori_loop(0, R, a_body, None)
        plsc.subcore_barrier()

        @pl.when(s == 0)
        def _():
            pltpu.sync_copy(deg_sh, deg_out.at[c])

    return scatter_ids, count_winners, NPAD


def _dense_kernel(x_ref, dp_ref, w1_ref, w2_ref, b1_ref, b2_ref,
                  t1_ref, t2_ref, o_ref):
    deg = dp_ref[0, :] + dp_ref[1, :]
    sd = jnp.sqrt(deg)
    c1 = jnp.cos(sd * t1_ref[0, 0])[:, None]
    c2 = jnp.cos(sd * t2_ref[0, 0])[:, None]
    h = lax.dot_general(x_ref[...], w1_ref[...], (((1,), (1,)), ((), ())),
                        preferred_element_type=jnp.float32)
    h = c1 * h + b1_ref[...]
    h = jnp.maximum(h, 0.0)
    h = lax.dot_general(h, w2_ref[...], (((1,), (1,)), ((), ())),
                        preferred_element_type=jnp.float32)
    h = c2 * h + b2_ref[...]
    m = jnp.max(h, axis=1, keepdims=True)
    ex = jnp.exp(h - m)
    sm = jnp.sum(ex, axis=1, keepdims=True)
    o_ref[...] = h - m - jnp.log(sm)


def kernel(x, edge_index, W1, b1, t1, W2, b2, t2):
    n = x.shape[0]
    d_in = x.shape[1]
    d_out = W2.shape[0]
    e = edge_index.shape[1]

    epw = e // NW
    assert epw * NW == e and epw % LANES == 0
    rows = (epw + 127) // 128
    full_rows = epw // 128
    rem_groups = (epw - full_rows * 128) // LANES
    if rem_groups == 0:
        rows += 1
    pads = rows * 128 - epw

    src = edge_index[0].astype(jnp.int32).reshape(NW, epw)
    dst = edge_index[1].astype(jnp.int32).reshape(NW, epw)
    padv = (jnp.arange(NW * pads, dtype=jnp.int32) * 997) % n
    padv = padv.reshape(NW, pads)
    src_p = jnp.concatenate([src, padv], axis=1).reshape(NW, rows, 128)
    dst_p = jnp.concatenate([dst, padv], axis=1).reshape(NW, rows, 128)

    scatter_ids, count_winners, npad = _edge_kernels(
        n, rows, full_rows, rem_groups, epw)
    table = scatter_ids(src_p, dst_p)
    deg_parts = count_winners(table, src_p, dst_p)

    x_p = jnp.zeros((npad, d_in), x.dtype).at[:n, :].set(x)
    rb = npad // 8
    grid = (npad // rb,)
    out = pl.pallas_call(
        _dense_kernel,
        grid=grid,
        in_specs=[
            pl.BlockSpec((rb, d_in), lambda i: (i, 0)),
            pl.BlockSpec((NC, rb), lambda i: (0, i)),
            pl.BlockSpec(W1.shape, lambda i: (0, 0)),
            pl.BlockSpec(W2.shape, lambda i: (0, 0)),
            pl.BlockSpec((1, d_in), lambda i: (0, 0)),
            pl.BlockSpec((1, d_out), lambda i: (0, 0)),
            pl.BlockSpec((1, 1), lambda i: (0, 0)),
            pl.BlockSpec((1, 1), lambda i: (0, 0)),
        ],
        out_specs=pl.BlockSpec((rb, d_out), lambda i: (i, 0)),
        out_shape=jax.ShapeDtypeStruct((npad, d_out), jnp.float32),
    )(x_p, deg_parts, W1, W2, b1.reshape(1, -1), b2.reshape(1, -1),
      jnp.reshape(t1, (1, 1)), jnp.reshape(t2, (1, 1)))
    return out[:n]


# speedup vs baseline: 3.0914x; 1.1311x over previous
import functools

import jax
import jax.numpy as jnp
from jax import lax
from jax.experimental import pallas as pl
from jax.experimental.pallas import tpu as pltpu
from jax.experimental.pallas import tpu_sc as plsc

NC = 2
NS = 16
NW = NC * NS
LANES = 16


def _edge_kernels(n_nodes, epad, full_groups, epw):
    N = n_nodes
    TBL = N * N
    PADS = epad - epw
    PGROUPS = PADS // LANES

    mesh = plsc.VectorSubcoreMesh(core_axis_name="c", subcore_axis_name="s")

    def compute_keys(wid, es_v, ed_v, key_v):
        iota = lax.iota(jnp.int32, LANES)

        def g_body(g, carry):
            a = es_v[pl.ds(g * LANES, LANES)]
            b = ed_v[pl.ds(g * LANES, LANES)]
            lo = jnp.minimum(a, b)
            hi = jnp.maximum(a, b)
            key_v[pl.ds(g * LANES, LANES)] = lo * N + hi
            return carry

        lax.fori_loop(0, full_groups, g_body, None)
        for j in range(PGROUPS):
            col = (full_groups + j) * LANES
            p0 = wid * PADS + j * LANES + 1
            key_v[pl.ds(col, LANES)] = (iota + p0) * N

    @functools.partial(
        pl.kernel,
        out_type=jax.ShapeDtypeStruct((TBL,), jnp.int32),
        mesh=mesh,
        scratch_types=[
            pltpu.VMEM((epad,), jnp.int32),
            pltpu.VMEM((epad,), jnp.int32),
            pltpu.VMEM((epad,), jnp.int32),
            pltpu.VMEM((epad,), jnp.int32),
            pltpu.SemaphoreType.DMA,
        ],
    )
    def scatter_ids(src_hbm, dst_hbm, table_out, es_v, ed_v, key_v, id_v, sem):
        c = lax.axis_index("c")
        s = lax.axis_index("s")
        wid = s * NC + c
        pltpu.sync_copy(src_hbm.at[wid], es_v)
        pltpu.sync_copy(dst_hbm.at[wid], ed_v)
        compute_keys(wid, es_v, ed_v, key_v)
        iota = lax.iota(jnp.int32, LANES)
        base_id = wid * epad

        def id_body(g, carry):
            id_v[pl.ds(g * LANES, LANES)] = base_id + g * LANES + iota
            return carry

        lax.fori_loop(0, epad // LANES, id_body, None)

        pltpu.async_copy(id_v, table_out.at[key_v], sem).wait()

    NPAD = ((N + 10 * LANES * NS - 1) // (10 * LANES * NS)) * (10 * LANES * NS)
    ZCH = NPAD // NS

    @functools.partial(
        pl.kernel,
        out_type=jax.ShapeDtypeStruct((NC, NPAD), jnp.float32),
        mesh=mesh,
        scratch_types=[
            pltpu.VMEM((epad,), jnp.int32),
            pltpu.VMEM((epad,), jnp.int32),
            pltpu.VMEM((epad,), jnp.int32),
            pltpu.VMEM((epad,), jnp.int32),
            pltpu.VMEM((epad,), jnp.float32),
            pltpu.VMEM((ZCH,), jnp.float32),
            pltpu.VMEM_SHARED((NPAD,), jnp.float32),
            pltpu.SemaphoreType.DMA,
        ],
    )
    def count_winners(table_hbm, src_hbm, dst_hbm, deg_out,
                      es_v, ed_v, key_v, got_v, w_v, z_v, deg_sh, sem):
        c = lax.axis_index("c")
        s = lax.axis_index("s")
        wid = s * NC + c
        pltpu.sync_copy(src_hbm.at[wid], es_v)
        pltpu.sync_copy(dst_hbm.at[wid], ed_v)
        compute_keys(wid, es_v, ed_v, key_v)

        gather = pltpu.async_copy(table_hbm.at[key_v], got_v, sem)

        def z_body(i, carry):
            z_v[pl.ds(i * LANES, LANES)] = jnp.zeros((LANES,), jnp.float32)
            return carry

        lax.fori_loop(0, ZCH // LANES, z_body, None)
        pltpu.sync_copy(z_v, deg_sh.at[pl.ds(s * ZCH, ZCH)])
        gather.wait()

        iota = lax.iota(jnp.int32, LANES)
        base_id = wid * epad
        one = jnp.full((LANES,), 1.0, jnp.float32)
        zero = jnp.zeros((LANES,), jnp.float32)

        def c_body(g, carry):
            myid = base_id + g * LANES + iota
            w_v[pl.ds(g * LANES, LANES)] = jnp.where(
                got_v[pl.ds(g * LANES, LANES)] == myid, one, zero)
            return carry

        lax.fori_loop(0, full_groups, c_body, None)
        for j in range(PGROUPS):
            col = (full_groups + j) * LANES
            w_v[pl.ds(col, LANES)] = zero

        plsc.subcore_barrier()
        add1 = pltpu.async_copy(w_v, deg_sh.at[es_v], sem, add=True)
        add2 = pltpu.async_copy(w_v, deg_sh.at[ed_v], sem, add=True)
        add1.wait()
        add2.wait()
        plsc.subcore_barrier()

        @pl.when(s == 0)
        def _():
            pltpu.sync_copy(deg_sh, deg_out.at[c])

    return scatter_ids, count_winners, NPAD


def _dense_kernel(x_ref, dp_ref, w1_ref, w2_ref, b1_ref, b2_ref,
                  t1_ref, t2_ref, o_ref):
    deg = dp_ref[0, :] + dp_ref[1, :]
    sd = jnp.sqrt(deg)
    c1 = jnp.cos(sd * t1_ref[0, 0])[:, None]
    c2 = jnp.cos(sd * t2_ref[0, 0])[:, None]
    h = lax.dot_general(x_ref[...], w1_ref[...], (((1,), (1,)), ((), ())),
                        preferred_element_type=jnp.float32)
    h = c1 * h + b1_ref[...]
    h = jnp.maximum(h, 0.0)
    h = lax.dot_general(h, w2_ref[...], (((1,), (1,)), ((), ())),
                        preferred_element_type=jnp.float32)
    h = c2 * h + b2_ref[...]
    m = jnp.max(h, axis=1, keepdims=True)
    ex = jnp.exp(h - m)
    sm = jnp.sum(ex, axis=1, keepdims=True)
    o_ref[...] = h - m - jnp.log(sm)


def kernel(x, edge_index, W1, b1, t1, W2, b2, t2):
    n = x.shape[0]
    d_in = x.shape[1]
    d_out = W2.shape[0]
    e = edge_index.shape[1]

    epw = e // NW
    assert epw * NW == e and epw % LANES == 0
    epad = ((epw + 127) // 128) * 128
    if epad == epw:
        epad += 128
    full_groups = epw // LANES
    pads = epad - epw

    src = edge_index[0].astype(jnp.int32).reshape(NW, epw)
    dst = edge_index[1].astype(jnp.int32).reshape(NW, epw)
    padv = (jnp.arange(NW * pads, dtype=jnp.int32) * 997) % n
    padv = padv.reshape(NW, pads)
    src_p = jnp.concatenate([src, padv], axis=1)
    dst_p = jnp.concatenate([dst, padv], axis=1)

    scatter_ids, count_winners, npad = _edge_kernels(n, epad, full_groups, epw)
    table = scatter_ids(src_p, dst_p)
    deg_parts = count_winners(table, src_p, dst_p)

    x_p = jnp.zeros((npad, d_in), x.dtype).at[:n, :].set(x)
    rb = npad // 8
    grid = (npad // rb,)
    out = pl.pallas_call(
        _dense_kernel,
        grid=grid,
        in_specs=[
            pl.BlockSpec((rb, d_in), lambda i: (i, 0)),
            pl.BlockSpec((NC, rb), lambda i: (0, i)),
            pl.BlockSpec(W1.shape, lambda i: (0, 0)),
            pl.BlockSpec(W2.shape, lambda i: (0, 0)),
            pl.BlockSpec((1, d_in), lambda i: (0, 0)),
            pl.BlockSpec((1, d_out), lambda i: (0, 0)),
            pl.BlockSpec((1, 1), lambda i: (0, 0)),
            pl.BlockSpec((1, 1), lambda i: (0, 0)),
        ],
        out_specs=pl.BlockSpec((rb, d_out), lambda i: (i, 0)),
        out_shape=jax.ShapeDtypeStruct((npad, d_out), jnp.float32),
    )(x_p, deg_parts, W1, W2, b1.reshape(1, -1), b2.reshape(1, -1),
      jnp.reshape(t1, (1, 1)), jnp.reshape(t2, (1, 1)))
    return out[:n]
